# baseline (speedup 1.0000x reference)
import jax
import jax.numpy as jnp
from jax import lax
from jax.experimental import pallas as pl
from jax.experimental.pallas import tpu as pltpu


_MIB = 1024 * 1024
_VMEM_LIMIT = 60 * _MIB
_DEPTH = 3
_BIG = 2560
_WARM = (512, 1024, 2048)


def _make_chunks(V):
    chunks = []
    off = 0
    for w in _WARM:
        if V - off > w:
            chunks.append((off, w))
            off += w
    while V - off >= _BIG:
        chunks.append((off, _BIG))
        off += _BIG
    if V - off > 0:
        chunks.append((off, V - off))
    return chunks


def _make_fused_kernel(chunks):
    n = len(chunks)

    def fused_kernel(x_ref, w1_ref, b1_ref, emb_ref, o_ref,
                     h_ref, ebuf, obuf, esem, osem):
        def e_copy(i):
            off, sz = chunks[i]
            return pltpu.make_async_copy(
                emb_ref.at[pl.ds(off, sz), :],
                ebuf.at[i % _DEPTH, pl.ds(0, sz), :],
                esem.at[i % _DEPTH])

        def o_copy(i):
            off, sz = chunks[i]
            return pltpu.make_async_copy(
                obuf.at[i % 2, :, pl.ds(0, sz)],
                o_ref.at[:, pl.ds(off, sz)],
                osem.at[i % 2])

        for i in range(min(_DEPTH, n)):
            e_copy(i).start()

        h = lax.dot_general(
            x_ref[...].astype(jnp.bfloat16), w1_ref[...].astype(jnp.bfloat16),
            dimension_numbers=(((1,), (1,)), ((), ())),
            preferred_element_type=jnp.float32)
        h_ref[...] = jnp.maximum(h + b1_ref[...], 0.0).astype(h_ref.dtype)

        for i in range(n):
            _, sz = chunks[i]
            e_copy(i).wait()
            part = lax.dot_general(
                h_ref[...], ebuf[i % _DEPTH, :sz, :].astype(jnp.bfloat16),
                dimension_numbers=(((1,), (1,)), ((), ())),
                preferred_element_type=jnp.float32)
            if i >= 2:
                o_copy(i - 2).wait()
            obuf[i % 2, :, :sz] = part.astype(obuf.dtype)
            o_copy(i).start()
            if i + _DEPTH < n:
                e_copy(i + _DEPTH).start()

        if n >= 2:
            o_copy(n - 2).wait()
        o_copy(n - 1).wait()

    return fused_kernel


def kernel(x, w1, b1, emb):
    B, S, D = x.shape
    V, D_e = emb.shape
    assert D_e == D
    M = B * S

    xm = x.reshape(M, D)
    b1_2d = b1.reshape(1, D)

    chunks = _make_chunks(V)
    tv_max = max(sz for _, sz in chunks)

    cost = pl.CostEstimate(
        flops=2 * M * D * (V + D),
        transcendentals=0,
        bytes_accessed=M * D * 4 + D * D * 4 + V * D * 4 + M * V * 4)

    out = pl.pallas_call(
        _make_fused_kernel(chunks),
        out_shape=jax.ShapeDtypeStruct((M, V), x.dtype),
        in_specs=[
            pl.BlockSpec((M, D), lambda: (0, 0)),
            pl.BlockSpec((D, D), lambda: (0, 0)),
            pl.BlockSpec((1, D), lambda: (0, 0)),
            pl.BlockSpec(memory_space=pl.ANY),
        ],
        out_specs=pl.BlockSpec(memory_space=pl.ANY),
        scratch_shapes=[
            pltpu.VMEM((M, D), jnp.bfloat16),
            pltpu.VMEM((_DEPTH, tv_max, D), jnp.float32),
            pltpu.VMEM((2, M, tv_max), jnp.float32),
            pltpu.SemaphoreType.DMA((_DEPTH,)),
            pltpu.SemaphoreType.DMA((2,)),
        ],
        compiler_params=pltpu.CompilerParams(
            vmem_limit_bytes=_VMEM_LIMIT),
        cost_estimate=cost,
    )(xm, w1, b1_2d, emb)

    return out.reshape(B, S, V)


# speedup vs baseline: 1.0179x; 1.0179x over previous
import jax
import jax.numpy as jnp
from jax import lax
from jax.experimental import pallas as pl
from jax.experimental.pallas import tpu as pltpu


_MIB = 1024 * 1024
_VMEM_LIMIT = 60 * _MIB
_DEPTH = 3
_BIG = 2560
_WARM = (1024, 2048)


def _make_chunks(V):
    chunks = []
    off = 0
    for w in _WARM:
        if V - off > w:
            chunks.append((off, w))
            off += w
    while V - off >= _BIG:
        chunks.append((off, _BIG))
        off += _BIG
    if V - off > 0:
        chunks.append((off, V - off))
    return chunks


def _make_fused_kernel(chunks):
    n = len(chunks)

    def fused_kernel(x_ref, w1_ref, b1_ref, emb_ref, o_ref,
                     h_ref, ebuf, obuf, esem, osem):
        def e_copy(i):
            off, sz = chunks[i]
            return pltpu.make_async_copy(
                emb_ref.at[pl.ds(off, sz), :],
                ebuf.at[i % _DEPTH, pl.ds(0, sz), :],
                esem.at[i % _DEPTH])

        def o_copy(i):
            off, sz = chunks[i]
            return pltpu.make_async_copy(
                obuf.at[i % 2, :, pl.ds(0, sz)],
                o_ref.at[:, pl.ds(off, sz)],
                osem.at[i % 2])

        for i in range(min(_DEPTH, n)):
            e_copy(i).start()

        h = lax.dot_general(
            x_ref[...].astype(jnp.bfloat16), w1_ref[...].astype(jnp.bfloat16),
            dimension_numbers=(((1,), (1,)), ((), ())),
            preferred_element_type=jnp.float32)
        h_ref[...] = jnp.maximum(h + b1_ref[...], 0.0).astype(h_ref.dtype)

        for i in range(n):
            _, sz = chunks[i]
            e_copy(i).wait()
            part = lax.dot_general(
                h_ref[...], ebuf[i % _DEPTH, :sz, :].astype(jnp.bfloat16),
                dimension_numbers=(((1,), (1,)), ((), ())),
                preferred_element_type=jnp.float32)
            if i >= 2:
                o_copy(i - 2).wait()
            obuf[i % 2, :, :sz] = part.astype(obuf.dtype)
            o_copy(i).start()
            if i + _DEPTH < n:
                e_copy(i + _DEPTH).start()

        if n >= 2:
            o_copy(n - 2).wait()
        o_copy(n - 1).wait()

    return fused_kernel


def kernel(x, w1, b1, emb):
    B, S, D = x.shape
    V, D_e = emb.shape
    assert D_e == D
    M = B * S

    xm = x.reshape(M, D)
    b1_2d = b1.reshape(1, D)

    chunks = _make_chunks(V)
    tv_max = max(sz for _, sz in chunks)

    cost = pl.CostEstimate(
        flops=2 * M * D * (V + D),
        transcendentals=0,
        bytes_accessed=M * D * 4 + D * D * 4 + V * D * 4 + M * V * 4)

    out = pl.pallas_call(
        _make_fused_kernel(chunks),
        out_shape=jax.ShapeDtypeStruct((M, V), x.dtype),
        in_specs=[
            pl.BlockSpec((M, D), lambda: (0, 0)),
            pl.BlockSpec((D, D), lambda: (0, 0)),
            pl.BlockSpec((1, D), lambda: (0, 0)),
            pl.BlockSpec(memory_space=pl.ANY),
        ],
        out_specs=pl.BlockSpec(memory_space=pl.ANY),
        scratch_shapes=[
            pltpu.VMEM((M, D), jnp.bfloat16),
            pltpu.VMEM((_DEPTH, tv_max, D), jnp.float32),
            pltpu.VMEM((2, M, tv_max), jnp.float32),
            pltpu.SemaphoreType.DMA((_DEPTH,)),
            pltpu.SemaphoreType.DMA((2,)),
        ],
        compiler_params=pltpu.CompilerParams(
            vmem_limit_bytes=_VMEM_LIMIT),
        cost_estimate=cost,
    )(xm, w1, b1_2d, emb)

    return out.reshape(B, S, V)
